# baseline (speedup 1.0000x reference)
import functools

import jax
import jax.numpy as jnp
from jax import lax
from jax.experimental import pallas as pl
from jax.experimental.pallas import tpu as pltpu
from jax.experimental.pallas import tpu_sc as plsc

N_NODES = 10000
N_EDGES = 320000
D = 128
N_GRAPHS = 64
N_CLASSES = 10
N_LAYERS = 5
BN_EPS = 1e-5

_NC = 2
_NS = 16
_NW = _NC * _NS
_C = 96
_NCH = 105
_EPW = _NCH * _C
_AGG_ROWS = 10008
_RPT = 640
_RPT_LAST = N_NODES - (_NS - 1) * _RPT


def _sc_scatter_body(h_hbm, eix_hbm, zeros_hbm, out_hbm,
                     src_v, rows_v, dst_b, agg_sh,
                     gs0, gs1, gs2, ds0, ds1, ds2):
    cid = lax.axis_index("c")
    sid = lax.axis_index("s")
    wid = sid * _NC + cid
    gsems = (gs0, gs1, gs2)
    dsems = (ds0, ds1, ds2)

    @pl.when(sid < _NS - 1)
    def _():
        pltpu.sync_copy(zeros_hbm.at[pl.ds(sid * _RPT, _RPT)],
                        agg_sh.at[pl.ds(sid * _RPT, _RPT)])

    @pl.when(sid == _NS - 1)
    def _():
        pltpu.sync_copy(zeros_hbm.at[pl.ds((_NS - 1) * _RPT, _RPT_LAST)],
                        agg_sh.at[pl.ds((_NS - 1) * _RPT, _RPT_LAST)])

    pltpu.sync_copy(eix_hbm.at[pl.ds(wid * _EPW, _EPW)], src_v)

    def _gather(j, b):
        return pltpu.async_copy(h_hbm.at[src_v.at[pl.ds(j * _C, _C)]],
                                rows_v.at[b], gsems[b])

    def _wait_gather(j, b):
        pltpu.make_async_copy(h_hbm.at[src_v.at[pl.ds(j * _C, _C)]],
                              rows_v.at[b], gsems[b]).wait()

    def _dst(j, b):
        return pltpu.async_copy(
            eix_hbm.at[pl.ds(_NW * _EPW + wid * _EPW + j * _C, _C)],
            dst_b.at[b], dsems[b])

    def _wait_dst(j, b):
        pltpu.make_async_copy(
            eix_hbm.at[pl.ds(_NW * _EPW + wid * _EPW + j * _C, _C)],
            dst_b.at[b], dsems[b]).wait()

    def _step(j, b):
        _gather(j + 2, (b + 2) % 3)
        _wait_gather(j, b)
        _wait_dst(j, b)
        pltpu.sync_copy(rows_v.at[b], agg_sh.at[dst_b.at[b]], add=True)

        @pl.when(j + 3 < _NCH)
        def _():
            _dst(j + 3, b)

    _dst(0, 0)
    _dst(1, 1)
    _dst(2, 2)
    _gather(0, 0)
    _gather(1, 1)
    plsc.subcore_barrier()

    def body(i, carry):
        jj = 3 * i
        _step(jj, 0)
        _step(jj + 1, 1)
        _step(jj + 2, 2)
        return carry

    lax.fori_loop(0, _NCH // 3 - 1, body, 0)

    _gather(_NCH - 1, (_NCH - 1) % 3)
    _wait_gather(_NCH - 3, 0)
    _wait_dst(_NCH - 3, 0)
    pltpu.sync_copy(rows_v.at[0], agg_sh.at[dst_b.at[0]], add=True)
    _wait_gather(_NCH - 2, 1)
    _wait_dst(_NCH - 2, 1)
    pltpu.sync_copy(rows_v.at[1], agg_sh.at[dst_b.at[1]], add=True)
    _wait_gather(_NCH - 1, 2)
    _wait_dst(_NCH - 1, 2)
    pltpu.sync_copy(rows_v.at[2], agg_sh.at[dst_b.at[2]], add=True)

    plsc.subcore_barrier()

    @pl.when(sid < _NS - 1)
    def _():
        pltpu.sync_copy(agg_sh.at[pl.ds(sid * _RPT, _RPT)],
                        out_hbm.at[cid, pl.ds(sid * _RPT, _RPT)])

    @pl.when(sid == _NS - 1)
    def _():
        pltpu.sync_copy(agg_sh.at[pl.ds((_NS - 1) * _RPT, _RPT_LAST)],
                        out_hbm.at[cid, pl.ds((_NS - 1) * _RPT, _RPT_LAST)])


_sc_scatter = pl.kernel(
    _sc_scatter_body,
    out_type=jax.ShapeDtypeStruct((_NC, N_NODES, D), jnp.float32),
    mesh=plsc.VectorSubcoreMesh(core_axis_name="c", subcore_axis_name="s"),
    scratch_types=[
        pltpu.VMEM((_EPW,), jnp.int32),
        pltpu.VMEM((3, _C, D), jnp.float32),
        pltpu.VMEM((3, _C), jnp.int32),
        pltpu.VMEM_SHARED((_AGG_ROWS, D), jnp.float32),
        pltpu.SemaphoreType.DMA,
        pltpu.SemaphoreType.DMA,
        pltpu.SemaphoreType.DMA,
        pltpu.SemaphoreType.DMA,
        pltpu.SemaphoreType.DMA,
        pltpu.SemaphoreType.DMA,
    ],
    name="gin_sc_scatter",
)


def _tc_layer_body(h_ref, parts_ref, eps1_ref, w1_ref, b1_ref,
                   w2_ref, b2_ref, g_ref, bt_ref, out_ref):
    hp = eps1_ref[0, 0] * h_ref[...] + (parts_ref[0] + parts_ref[1])
    y = jnp.dot(hp, w1_ref[...], preferred_element_type=jnp.float32)
    y = jnp.maximum(y + b1_ref[...], 0.0)
    z = jnp.dot(y, w2_ref[...], preferred_element_type=jnp.float32)
    z = z + b2_ref[...]
    mean = jnp.mean(z, axis=0, keepdims=True)
    zc = z - mean
    var = jnp.mean(zc * zc, axis=0, keepdims=True)
    zn = zc * lax.rsqrt(var + BN_EPS) * g_ref[...] + bt_ref[...]
    out_ref[...] = jnp.maximum(zn, 0.0)


_tc_layer = pl.pallas_call(
    _tc_layer_body,
    out_shape=jax.ShapeDtypeStruct((N_NODES, D), jnp.float32),
    compiler_params=pltpu.CompilerParams(vmem_limit_bytes=100 * 1024 * 1024),
)


def _tc_last_body(h_ref, parts_ref, eps1_ref, w1_ref, b1_ref,
                  w2_ref, b2_ref, g_ref, bt_ref, batch_ref,
                  cw1_ref, cb1_ref, cw2_ref, cb2_ref, out_ref):
    hp = eps1_ref[0, 0] * h_ref[...] + (parts_ref[0] + parts_ref[1])
    y = jnp.dot(hp, w1_ref[...], preferred_element_type=jnp.float32)
    y = jnp.maximum(y + b1_ref[...], 0.0)
    z = jnp.dot(y, w2_ref[...], preferred_element_type=jnp.float32)
    z = z + b2_ref[...]
    mean = jnp.mean(z, axis=0, keepdims=True)
    zc = z - mean
    var = jnp.mean(zc * zc, axis=0, keepdims=True)
    zn = zc * lax.rsqrt(var + BN_EPS) * g_ref[...] + bt_ref[...]
    h = jnp.maximum(zn, 0.0)
    seg = lax.broadcasted_iota(jnp.int32, (N_GRAPHS, N_NODES), 0)
    onehot = jnp.where(seg == batch_ref[...], 1.0, 0.0)
    hg = jnp.dot(onehot, h, preferred_element_type=jnp.float32)
    c = jnp.dot(hg, cw1_ref[...], preferred_element_type=jnp.float32)
    c = jnp.maximum(c + cb1_ref[...], 0.0)
    out_ref[...] = jnp.dot(c, cw2_ref[...],
                           preferred_element_type=jnp.float32) + cb2_ref[...]


_tc_last = pl.pallas_call(
    _tc_last_body,
    out_shape=jax.ShapeDtypeStruct((N_GRAPHS, N_CLASSES), jnp.float32),
    compiler_params=pltpu.CompilerParams(vmem_limit_bytes=100 * 1024 * 1024),
)


def kernel(x, edge_index, batch, params):
    edge_index = edge_index.astype(jnp.int32)
    pad = _EPW - N_EDGES // _NW
    s = jnp.pad(edge_index[0].reshape(_NW, N_EDGES // _NW), ((0, 0), (0, pad)))
    d = jnp.pad(edge_index[1].reshape(_NW, N_EDGES // _NW), ((0, 0), (0, pad)),
                constant_values=N_NODES)
    eix = jnp.concatenate([s.reshape(-1), d.reshape(-1)])
    zeros = jnp.zeros((N_NODES, D), jnp.float32)

    h = x
    for i in range(N_LAYERS - 1):
        parts = _sc_scatter(h, eix, zeros)
        eps1 = (1.0 + params[f"eps_{i}"]).reshape(1, 1)
        h = _tc_layer(h, parts, eps1,
                      params[f"w1_{i}"], params[f"b1_{i}"].reshape(1, D),
                      params[f"w2_{i}"], params[f"b2_{i}"].reshape(1, D),
                      params[f"gamma_{i}"].reshape(1, D),
                      params[f"beta_{i}"].reshape(1, D))

    i = N_LAYERS - 1
    parts = _sc_scatter(h, eix, zeros)
    eps1 = (1.0 + params[f"eps_{i}"]).reshape(1, 1)
    logits = _tc_last(h, parts, eps1,
                      params[f"w1_{i}"], params[f"b1_{i}"].reshape(1, D),
                      params[f"w2_{i}"], params[f"b2_{i}"].reshape(1, D),
                      params[f"gamma_{i}"].reshape(1, D),
                      params[f"beta_{i}"].reshape(1, D),
                      batch.astype(jnp.int32).reshape(1, N_NODES),
                      params["cls_w1"], params["cls_b1"].reshape(1, D),
                      params["cls_w2"],
                      params["cls_b2"].reshape(1, N_CLASSES))
    return logits


# speedup vs baseline: 1.8505x; 1.8505x over previous
import functools

import jax
import jax.numpy as jnp
from jax import lax
from jax.experimental import pallas as pl
from jax.experimental.pallas import tpu as pltpu
from jax.experimental.pallas import tpu_sc as plsc

N_NODES = 10000
N_EDGES = 320000
D = 128
N_GRAPHS = 64
N_CLASSES = 10
N_LAYERS = 5
BN_EPS = 1e-5

_NC = 2
_NS = 16
_NW = _NC * _NS
_EPW = N_EDGES // _NW
_C = 80
_NCH = _EPW // _C
_RPT = 640
_RPT_LAST = N_NODES - (_NS - 1) * _RPT


def _sc_scatter_body(h_hbm, eix_hbm, zeros_hbm, out_hbm,
                     src_v, rows_v, dst_b, agg_sh,
                     gs0, gs1, gs2, ds0, ds1, ds2):
    cid = lax.axis_index("c")
    sid = lax.axis_index("s")
    wid = sid * _NC + cid
    gsems = (gs0, gs1, gs2)
    dsems = (ds0, ds1, ds2)

    @pl.when(sid < _NS - 1)
    def _():
        pltpu.sync_copy(zeros_hbm.at[pl.ds(sid * _RPT, _RPT)],
                        agg_sh.at[pl.ds(sid * _RPT, _RPT)])

    @pl.when(sid == _NS - 1)
    def _():
        pltpu.sync_copy(zeros_hbm.at[pl.ds((_NS - 1) * _RPT, _RPT_LAST)],
                        agg_sh.at[pl.ds((_NS - 1) * _RPT, _RPT_LAST)])

    pltpu.sync_copy(eix_hbm.at[pl.ds(wid * _EPW, _EPW)], src_v)

    def _gather(j, b):
        return pltpu.async_copy(h_hbm.at[src_v.at[pl.ds(j * _C, _C)]],
                                rows_v.at[b], gsems[b])

    def _wait_gather(j, b):
        pltpu.make_async_copy(h_hbm.at[src_v.at[pl.ds(j * _C, _C)]],
                              rows_v.at[b], gsems[b]).wait()

    def _dst(j, b):
        return pltpu.async_copy(
            eix_hbm.at[pl.ds(N_EDGES + wid * _EPW + j * _C, _C)],
            dst_b.at[b], dsems[b])

    def _wait_dst(j, b):
        pltpu.make_async_copy(
            eix_hbm.at[pl.ds(N_EDGES + wid * _EPW + j * _C, _C)],
            dst_b.at[b], dsems[b]).wait()

    def _step(j, b):
        _gather(j + 2, (b + 2) % 3)
        _wait_gather(j, b)
        _wait_dst(j, b)
        pltpu.sync_copy(rows_v.at[b], agg_sh.at[dst_b.at[b]], add=True)

        @pl.when(j + 3 < _NCH)
        def _():
            _dst(j + 3, b)

    _dst(0, 0)
    _dst(1, 1)
    _dst(2, 2)
    _gather(0, 0)
    _gather(1, 1)
    plsc.subcore_barrier()

    def body(i, carry):
        jj = 3 * i
        _step(jj, 0)
        _step(jj + 1, 1)
        _step(jj + 2, 2)
        return carry

    lax.fori_loop(0, _NCH // 3, body, 0)

    _wait_gather(_NCH - 2, 0)
    _wait_dst(_NCH - 2, 0)
    pltpu.sync_copy(rows_v.at[0], agg_sh.at[dst_b.at[0]], add=True)
    _wait_gather(_NCH - 1, 1)
    _wait_dst(_NCH - 1, 1)
    pltpu.sync_copy(rows_v.at[1], agg_sh.at[dst_b.at[1]], add=True)

    plsc.subcore_barrier()

    @pl.when(sid < _NS - 1)
    def _():
        pltpu.sync_copy(agg_sh.at[pl.ds(sid * _RPT, _RPT)],
                        out_hbm.at[cid, pl.ds(sid * _RPT, _RPT)])

    @pl.when(sid == _NS - 1)
    def _():
        pltpu.sync_copy(agg_sh.at[pl.ds((_NS - 1) * _RPT, _RPT_LAST)],
                        out_hbm.at[cid, pl.ds((_NS - 1) * _RPT, _RPT_LAST)])


_sc_scatter = pl.kernel(
    _sc_scatter_body,
    out_type=jax.ShapeDtypeStruct((_NC, N_NODES, D), jnp.float32),
    mesh=plsc.VectorSubcoreMesh(core_axis_name="c", subcore_axis_name="s"),
    scratch_types=[
        pltpu.VMEM((_EPW,), jnp.int32),
        pltpu.VMEM((3, _C, D), jnp.float32),
        pltpu.VMEM((3, _C), jnp.int32),
        pltpu.VMEM_SHARED((N_NODES, D), jnp.float32),
        pltpu.SemaphoreType.DMA,
        pltpu.SemaphoreType.DMA,
        pltpu.SemaphoreType.DMA,
        pltpu.SemaphoreType.DMA,
        pltpu.SemaphoreType.DMA,
        pltpu.SemaphoreType.DMA,
    ],
    name="gin_sc_scatter",
)


def _tc_layer_body(h_ref, parts_ref, eps1_ref, w1_ref, b1_ref,
                   w2_ref, b2_ref, g_ref, bt_ref, out_ref):
    hp = eps1_ref[0, 0] * h_ref[...] + (parts_ref[0] + parts_ref[1])
    y = jnp.dot(hp, w1_ref[...], preferred_element_type=jnp.float32)
    y = jnp.maximum(y + b1_ref[...], 0.0)
    z = jnp.dot(y, w2_ref[...], preferred_element_type=jnp.float32)
    z = z + b2_ref[...]
    mean = jnp.mean(z, axis=0, keepdims=True)
    zc = z - mean
    var = jnp.mean(zc * zc, axis=0, keepdims=True)
    zn = zc * lax.rsqrt(var + BN_EPS) * g_ref[...] + bt_ref[...]
    out_ref[...] = jnp.maximum(zn, 0.0)


_tc_layer = pl.pallas_call(
    _tc_layer_body,
    out_shape=jax.ShapeDtypeStruct((N_NODES, D), jnp.float32),
    compiler_params=pltpu.CompilerParams(vmem_limit_bytes=100 * 1024 * 1024),
)


def _tc_last_body(h_ref, parts_ref, eps1_ref, w1_ref, b1_ref,
                  w2_ref, b2_ref, g_ref, bt_ref, batch_ref,
                  cw1_ref, cb1_ref, cw2_ref, cb2_ref, out_ref):
    hp = eps1_ref[0, 0] * h_ref[...] + (parts_ref[0] + parts_ref[1])
    y = jnp.dot(hp, w1_ref[...], preferred_element_type=jnp.float32)
    y = jnp.maximum(y + b1_ref[...], 0.0)
    z = jnp.dot(y, w2_ref[...], preferred_element_type=jnp.float32)
    z = z + b2_ref[...]
    mean = jnp.mean(z, axis=0, keepdims=True)
    zc = z - mean
    var = jnp.mean(zc * zc, axis=0, keepdims=True)
    zn = zc * lax.rsqrt(var + BN_EPS) * g_ref[...] + bt_ref[...]
    h = jnp.maximum(zn, 0.0)
    seg = lax.broadcasted_iota(jnp.int32, (N_GRAPHS, N_NODES), 0)
    onehot = jnp.where(seg == batch_ref[...], 1.0, 0.0)
    hg = jnp.dot(onehot, h, preferred_element_type=jnp.float32)
    c = jnp.dot(hg, cw1_ref[...], preferred_element_type=jnp.float32)
    c = jnp.maximum(c + cb1_ref[...], 0.0)
    out_ref[...] = jnp.dot(c, cw2_ref[...],
                           preferred_element_type=jnp.float32) + cb2_ref[...]


_tc_last = pl.pallas_call(
    _tc_last_body,
    out_shape=jax.ShapeDtypeStruct((N_GRAPHS, N_CLASSES), jnp.float32),
    compiler_params=pltpu.CompilerParams(vmem_limit_bytes=100 * 1024 * 1024),
)


def kernel(x, edge_index, batch, params):
    eix = edge_index.astype(jnp.int32).reshape(2 * N_EDGES)
    zeros = jnp.zeros((N_NODES, D), jnp.float32)

    h = x
    for i in range(N_LAYERS - 1):
        parts = _sc_scatter(h, eix, zeros)
        eps1 = (1.0 + params[f"eps_{i}"]).reshape(1, 1)
        h = _tc_layer(h, parts, eps1,
                      params[f"w1_{i}"], params[f"b1_{i}"].reshape(1, D),
                      params[f"w2_{i}"], params[f"b2_{i}"].reshape(1, D),
                      params[f"gamma_{i}"].reshape(1, D),
                      params[f"beta_{i}"].reshape(1, D))

    i = N_LAYERS - 1
    parts = _sc_scatter(h, eix, zeros)
    eps1 = (1.0 + params[f"eps_{i}"]).reshape(1, 1)
    logits = _tc_last(h, parts, eps1,
                      params[f"w1_{i}"], params[f"b1_{i}"].reshape(1, D),
                      params[f"w2_{i}"], params[f"b2_{i}"].reshape(1, D),
                      params[f"gamma_{i}"].reshape(1, D),
                      params[f"beta_{i}"].reshape(1, D),
                      batch.astype(jnp.int32).reshape(1, N_NODES),
                      params["cls_w1"], params["cls_b1"].reshape(1, D),
                      params["cls_w2"],
                      params["cls_b2"].reshape(1, N_CLASSES))
    return logits


# speedup vs baseline: 1.8819x; 1.0170x over previous
import functools

import jax
import jax.numpy as jnp
from jax import lax
from jax.experimental import pallas as pl
from jax.experimental.pallas import tpu as pltpu
from jax.experimental.pallas import tpu_sc as plsc

N_NODES = 10000
N_EDGES = 320000
D = 128
N_GRAPHS = 64
N_CLASSES = 10
N_LAYERS = 5
BN_EPS = 1e-5

_NC = 2
_NS = 16
_NW = _NC * _NS
_EPW = N_EDGES // _NW
_C = 80
_NCH = _EPW // _C
_RPT = 640
_RPT_LAST = N_NODES - (_NS - 1) * _RPT


def _sc_scatter_body(h_hbm, eix_hbm, zeros_hbm, out_hbm,
                     src_v, rows_v, dst_b, agg_sh,
                     gs0, gs1, gs2, ds0, ds1, ds2, zsem):
    cid = lax.axis_index("c")
    sid = lax.axis_index("s")
    wid = sid * _NC + cid
    gsems = (gs0, gs1, gs2)
    dsems = (ds0, ds1, ds2)

    @pl.when(sid < _NS - 1)
    def _():
        pltpu.async_copy(zeros_hbm.at[pl.ds(sid * _RPT, _RPT)],
                         agg_sh.at[pl.ds(sid * _RPT, _RPT)], zsem)

    @pl.when(sid == _NS - 1)
    def _():
        pltpu.async_copy(zeros_hbm.at[pl.ds((_NS - 1) * _RPT, _RPT_LAST)],
                         agg_sh.at[pl.ds((_NS - 1) * _RPT, _RPT_LAST)], zsem)

    pltpu.sync_copy(eix_hbm.at[pl.ds(wid * _EPW, _EPW)], src_v)

    def _gather(j, b):
        return pltpu.async_copy(h_hbm.at[src_v.at[pl.ds(j * _C, _C)]],
                                rows_v.at[b], gsems[b])

    def _wait_gather(j, b):
        pltpu.make_async_copy(h_hbm.at[src_v.at[pl.ds(j * _C, _C)]],
                              rows_v.at[b], gsems[b]).wait()

    def _dst(j, b):
        return pltpu.async_copy(
            eix_hbm.at[pl.ds(N_EDGES + wid * _EPW + j * _C, _C)],
            dst_b.at[b], dsems[b])

    def _wait_dst(j, b):
        pltpu.make_async_copy(
            eix_hbm.at[pl.ds(N_EDGES + wid * _EPW + j * _C, _C)],
            dst_b.at[b], dsems[b]).wait()

    def _step(j, b):
        _gather(j + 2, (b + 2) % 3)
        _wait_gather(j, b)
        _wait_dst(j, b)
        pltpu.sync_copy(rows_v.at[b], agg_sh.at[dst_b.at[b]], add=True)

        @pl.when(j + 3 < _NCH)
        def _():
            _dst(j + 3, b)

    _dst(0, 0)
    _dst(1, 1)
    _dst(2, 2)
    _gather(0, 0)
    _gather(1, 1)

    @pl.when(sid < _NS - 1)
    def _():
        pltpu.make_async_copy(zeros_hbm.at[pl.ds(sid * _RPT, _RPT)],
                              agg_sh.at[pl.ds(sid * _RPT, _RPT)],
                              zsem).wait()

    @pl.when(sid == _NS - 1)
    def _():
        pltpu.make_async_copy(
            zeros_hbm.at[pl.ds((_NS - 1) * _RPT, _RPT_LAST)],
            agg_sh.at[pl.ds((_NS - 1) * _RPT, _RPT_LAST)], zsem).wait()

    plsc.subcore_barrier()

    def body(i, carry):
        jj = 3 * i
        _step(jj, 0)
        _step(jj + 1, 1)
        _step(jj + 2, 2)
        return carry

    lax.fori_loop(0, _NCH // 3, body, 0)

    _wait_gather(_NCH - 2, 0)
    _wait_dst(_NCH - 2, 0)
    pltpu.sync_copy(rows_v.at[0], agg_sh.at[dst_b.at[0]], add=True)
    _wait_gather(_NCH - 1, 1)
    _wait_dst(_NCH - 1, 1)
    pltpu.sync_copy(rows_v.at[1], agg_sh.at[dst_b.at[1]], add=True)

    plsc.subcore_barrier()

    @pl.when(sid < _NS - 1)
    def _():
        pltpu.sync_copy(agg_sh.at[pl.ds(sid * _RPT, _RPT)],
                        out_hbm.at[cid, pl.ds(sid * _RPT, _RPT)])

    @pl.when(sid == _NS - 1)
    def _():
        pltpu.sync_copy(agg_sh.at[pl.ds((_NS - 1) * _RPT, _RPT_LAST)],
                        out_hbm.at[cid, pl.ds((_NS - 1) * _RPT, _RPT_LAST)])


_sc_scatter = pl.kernel(
    _sc_scatter_body,
    out_type=jax.ShapeDtypeStruct((_NC, N_NODES, D), jnp.float32),
    mesh=plsc.VectorSubcoreMesh(core_axis_name="c", subcore_axis_name="s"),
    scratch_types=[
        pltpu.VMEM((_EPW,), jnp.int32),
        pltpu.VMEM((3, _C, D), jnp.float32),
        pltpu.VMEM((3, _C), jnp.int32),
        pltpu.VMEM_SHARED((N_NODES, D), jnp.float32),
        pltpu.SemaphoreType.DMA,
        pltpu.SemaphoreType.DMA,
        pltpu.SemaphoreType.DMA,
        pltpu.SemaphoreType.DMA,
        pltpu.SemaphoreType.DMA,
        pltpu.SemaphoreType.DMA,
        pltpu.SemaphoreType.DMA,
    ],
    name="gin_sc_scatter",
)


def _tc_layer_body(h_ref, parts_ref, eps1_ref, w1_ref, b1_ref,
                   w2_ref, b2_ref, g_ref, bt_ref, out_ref):
    hp = eps1_ref[0, 0] * h_ref[...] + (parts_ref[0] + parts_ref[1])
    y = jnp.dot(hp, w1_ref[...], preferred_element_type=jnp.float32)
    y = jnp.maximum(y + b1_ref[...], 0.0)
    z = jnp.dot(y, w2_ref[...], preferred_element_type=jnp.float32)
    z = z + b2_ref[...]
    mean = jnp.mean(z, axis=0, keepdims=True)
    zc = z - mean
    var = jnp.mean(zc * zc, axis=0, keepdims=True)
    zn = zc * lax.rsqrt(var + BN_EPS) * g_ref[...] + bt_ref[...]
    out_ref[...] = jnp.maximum(zn, 0.0)


_tc_layer = pl.pallas_call(
    _tc_layer_body,
    out_shape=jax.ShapeDtypeStruct((N_NODES, D), jnp.float32),
    compiler_params=pltpu.CompilerParams(vmem_limit_bytes=100 * 1024 * 1024),
)


def _tc_last_body(h_ref, parts_ref, eps1_ref, w1_ref, b1_ref,
                  w2_ref, b2_ref, g_ref, bt_ref, batch_ref,
                  cw1_ref, cb1_ref, cw2_ref, cb2_ref, out_ref):
    hp = eps1_ref[0, 0] * h_ref[...] + (parts_ref[0] + parts_ref[1])
    y = jnp.dot(hp, w1_ref[...], preferred_element_type=jnp.float32)
    y = jnp.maximum(y + b1_ref[...], 0.0)
    z = jnp.dot(y, w2_ref[...], preferred_element_type=jnp.float32)
    z = z + b2_ref[...]
    mean = jnp.mean(z, axis=0, keepdims=True)
    zc = z - mean
    var = jnp.mean(zc * zc, axis=0, keepdims=True)
    zn = zc * lax.rsqrt(var + BN_EPS) * g_ref[...] + bt_ref[...]
    h = jnp.maximum(zn, 0.0)
    seg = lax.broadcasted_iota(jnp.int32, (N_GRAPHS, N_NODES), 0)
    onehot = jnp.where(seg == batch_ref[...], 1.0, 0.0)
    hg = jnp.dot(onehot, h, preferred_element_type=jnp.float32)
    c = jnp.dot(hg, cw1_ref[...], preferred_element_type=jnp.float32)
    c = jnp.maximum(c + cb1_ref[...], 0.0)
    out_ref[...] = jnp.dot(c, cw2_ref[...],
                           preferred_element_type=jnp.float32) + cb2_ref[...]


_tc_last = pl.pallas_call(
    _tc_last_body,
    out_shape=jax.ShapeDtypeStruct((N_GRAPHS, N_CLASSES), jnp.float32),
    compiler_params=pltpu.CompilerParams(vmem_limit_bytes=100 * 1024 * 1024),
)


def kernel(x, edge_index, batch, params):
    eix = edge_index.astype(jnp.int32).reshape(2 * N_EDGES)
    zeros = jnp.zeros((N_NODES, D), jnp.float32)

    h = x
    for i in range(N_LAYERS - 1):
        parts = _sc_scatter(h, eix, zeros)
        eps1 = (1.0 + params[f"eps_{i}"]).reshape(1, 1)
        h = _tc_layer(h, parts, eps1,
                      params[f"w1_{i}"], params[f"b1_{i}"].reshape(1, D),
                      params[f"w2_{i}"], params[f"b2_{i}"].reshape(1, D),
                      params[f"gamma_{i}"].reshape(1, D),
                      params[f"beta_{i}"].reshape(1, D))

    i = N_LAYERS - 1
    parts = _sc_scatter(h, eix, zeros)
    eps1 = (1.0 + params[f"eps_{i}"]).reshape(1, 1)
    logits = _tc_last(h, parts, eps1,
                      params[f"w1_{i}"], params[f"b1_{i}"].reshape(1, D),
                      params[f"w2_{i}"], params[f"b2_{i}"].reshape(1, D),
                      params[f"gamma_{i}"].reshape(1, D),
                      params[f"beta_{i}"].reshape(1, D),
                      batch.astype(jnp.int32).reshape(1, N_NODES),
                      params["cls_w1"], params["cls_b1"].reshape(1, D),
                      params["cls_w2"],
                      params["cls_b2"].reshape(1, N_CLASSES))
    return logits
